# Initial kernel scaffold; baseline (speedup 1.0000x reference)
import jax
import jax.numpy as jnp
from jax.experimental import pallas as pl


def kernel(target, contexts, negatives, w_emb, C_emb):
    raise NotImplementedError("write your pallas kernel here")


# speedup vs baseline: 8.8331x; 8.8331x over previous
import functools

import jax
import jax.numpy as jnp
from jax import lax
from jax.experimental import pallas as pl
from jax.experimental.pallas import tpu as pltpu
from jax.experimental.pallas import tpu_sc as plsc

V = 100000
D = 64
B = 16384
L = 20

NC = 2
NS = 16
NW = NC * NS
NB = B // NW
CB = 32
NCHUNK = NB // CB
ROWS = CB * L
GCALLS = ROWS // 128


def _sc_body(target_hbm, ctx_hbm, neg_hbm, w_hbm, c_hbm,
             pos_out, neg_out,
             cidx, nidx, tidx, cbuf, nbuf, tbuf, pstage, nstage, sem):
  wid = lax.axis_index("s") * NC + lax.axis_index("c")

  def chunk_body(g, _):
    base = wid * NB + g * CB

    pltpu.sync_copy(ctx_hbm.at[pl.ds(base * L, ROWS)], cidx)
    pltpu.sync_copy(neg_hbm.at[pl.ds(base * L, ROWS)], nidx)
    pltpu.sync_copy(target_hbm.at[pl.ds(base, CB)], tidx)

    cps = []
    for j in range(GCALLS):
      cps.append(pltpu.async_copy(
          c_hbm.at[cidx.at[pl.ds(j * 128, 128)]],
          cbuf.at[pl.ds(j * 128, 128)], sem))
      cps.append(pltpu.async_copy(
          c_hbm.at[nidx.at[pl.ds(j * 128, 128)]],
          nbuf.at[pl.ds(j * 128, 128)], sem))
    cps.append(pltpu.async_copy(w_hbm.at[tidx], tbuf, sem))
    for cp in cps:
      cp.wait()

    def elem_body(b, _):
      r0 = b * L
      pvec = jnp.zeros((16,), jnp.float32)
      nvec = jnp.zeros((16,), jnp.float32)
      for j in range(D // 16):
        sl = pl.ds(j * 16, 16)
        ca = cbuf[r0, sl]
        na = nbuf[r0, sl]
        for l in range(1, L):
          ca = ca + cbuf[r0 + l, sl]
          na = na + nbuf[r0 + l, sl]
        t = tbuf[b, sl]
        pvec = pvec + t * ca
        nvec = nvec + t * na
      pstage[b, :] = pvec
      nstage[b, :] = nvec
      return 0

    lax.fori_loop(0, CB, elem_body, 0)
    pltpu.sync_copy(pstage, pos_out.at[pl.ds(base, CB)])
    pltpu.sync_copy(nstage, neg_out.at[pl.ds(base, CB)])
    return 0

  lax.fori_loop(0, NCHUNK, chunk_body, 0)


_sc_call = functools.partial(
    pl.kernel,
    out_type=[jax.ShapeDtypeStruct((B, 16), jnp.float32),
              jax.ShapeDtypeStruct((B, 16), jnp.float32)],
    mesh=plsc.VectorSubcoreMesh(core_axis_name="c", subcore_axis_name="s"),
    compiler_params=pltpu.CompilerParams(use_tc_tiling_on_sc=False),
    scratch_types=[
        pltpu.VMEM((ROWS,), jnp.int32),
        pltpu.VMEM((ROWS,), jnp.int32),
        pltpu.VMEM((CB,), jnp.int32),
        pltpu.VMEM((ROWS, D), jnp.float32),
        pltpu.VMEM((ROWS, D), jnp.float32),
        pltpu.VMEM((CB, D), jnp.float32),
        pltpu.VMEM((CB, 16), jnp.float32),
        pltpu.VMEM((CB, 16), jnp.float32),
        pltpu.SemaphoreType.DMA,
    ],
)(_sc_body)


def _tc_body(p_ref, n_ref, o_ref):
  pos = jnp.sum(p_ref[...], axis=1, keepdims=True)
  neg = jnp.sum(n_ref[...], axis=1, keepdims=True)
  pos = jnp.clip(pos, -10.0, 10.0)
  neg = jnp.clip(neg, -10.0, 10.0)
  loss = jnp.log1p(jnp.exp(-pos)) + jnp.log1p(jnp.exp(neg))
  o_ref[...] = (jnp.sum(loss) / B).reshape(1, 1)


_tc_call = pl.pallas_call(
    _tc_body,
    out_shape=jax.ShapeDtypeStruct((1, 1), jnp.float32),
)


@jax.jit
def kernel(target, contexts, negatives, w_emb, C_emb):
  target = jnp.asarray(target, jnp.int32)
  ctx_flat = jnp.asarray(contexts, jnp.int32).reshape(B * L)
  neg_flat = jnp.asarray(negatives, jnp.int32).reshape(B * L)
  pos_part, neg_part = _sc_call(target, ctx_flat, neg_flat, w_emb, C_emb)
  return _tc_call(pos_part, neg_part)[0, 0]


# speedup vs baseline: 10.0600x; 1.1389x over previous
import functools

import jax
import jax.numpy as jnp
from jax import lax
from jax.experimental import pallas as pl
from jax.experimental.pallas import tpu as pltpu
from jax.experimental.pallas import tpu_sc as plsc

V = 100000
D = 64
B = 16384
L = 20

NC = 2
NS = 16
NW = NC * NS
NB = B // NW
CB = 32
NCHUNK = NB // CB
ROWS = CB * L
GROWS = 128
GCALLS = ROWS // GROWS


def _sc_body(target_hbm, ctx_hbm, neg_hbm, w_hbm, c_hbm,
             pos_out, neg_out,
             cidx, nidx, tidx, cbuf, nbuf, tbuf, pstage, nstage, sems):
  wid = lax.axis_index("s") * NC + lax.axis_index("c")

  def slot_refs(s):
    return (cidx[s], nidx[s], tidx[s], cbuf[s], nbuf[s], tbuf[s],
            pstage[s], nstage[s], sems[s])

  def gather_copies(s):
    ci, ni, ti, cb, nb, tb, _, _, sem = slot_refs(s)
    cps = []
    for j in range(GCALLS):
      sl = pl.ds(j * GROWS, GROWS)
      cps.append(pltpu.make_async_copy(c_hbm.at[ci.at[sl]], cb.at[sl], sem))
      cps.append(pltpu.make_async_copy(c_hbm.at[ni.at[sl]], nb.at[sl], sem))
    cps.append(pltpu.make_async_copy(w_hbm.at[ti], tb, sem))
    return cps

  def issue(s, g):
    ci, ni, ti, _, _, _, _, _, _ = slot_refs(s)
    base = wid * NB + g * CB
    pltpu.sync_copy(ctx_hbm.at[pl.ds(base * L, ROWS)], ci)
    pltpu.sync_copy(neg_hbm.at[pl.ds(base * L, ROWS)], ni)
    pltpu.sync_copy(target_hbm.at[pl.ds(base, CB)], ti)
    for cp in gather_copies(s):
      cp.start()

  def compute(s, g):
    _, _, _, cb, nb, tb, ps, ns, _ = slot_refs(s)
    base = wid * NB + g * CB
    for cp in gather_copies(s):
      cp.wait()

    def elem_body(b, _):
      r0 = b * L
      lo = pl.ds(0, 32)
      hi = pl.ds(32, 32)
      ca0 = cb[r0, lo]
      ca1 = cb[r0, hi]
      na0 = nb[r0, lo]
      na1 = nb[r0, hi]
      for l in range(1, L):
        ca0 = ca0 + cb[r0 + l, lo]
        ca1 = ca1 + cb[r0 + l, hi]
        na0 = na0 + nb[r0 + l, lo]
        na1 = na1 + nb[r0 + l, hi]
      t0 = tb[b, lo]
      t1 = tb[b, hi]
      ps[b, :] = t0 * ca0 + t1 * ca1
      ns[b, :] = t0 * na0 + t1 * na1
      return 0

    lax.fori_loop(0, CB, elem_body, 0)
    pltpu.sync_copy(ps, pos_out.at[pl.ds(base, CB)])
    pltpu.sync_copy(ns, neg_out.at[pl.ds(base, CB)])

  issue(0, 0)
  issue(1, 1)

  def pair_body(i, _):
    g0 = 2 * i
    compute(0, g0)

    @pl.when(i < NCHUNK // 2 - 1)
    def _():
      issue(0, g0 + 2)

    compute(1, g0 + 1)

    @pl.when(i < NCHUNK // 2 - 1)
    def _():
      issue(1, g0 + 3)

    return 0

  lax.fori_loop(0, NCHUNK // 2, pair_body, 0)


_sc_call = functools.partial(
    pl.kernel,
    out_type=[jax.ShapeDtypeStruct((B, 32), jnp.bfloat16),
              jax.ShapeDtypeStruct((B, 32), jnp.bfloat16)],
    mesh=plsc.VectorSubcoreMesh(core_axis_name="c", subcore_axis_name="s"),
    compiler_params=pltpu.CompilerParams(use_tc_tiling_on_sc=False),
    scratch_types=[
        [pltpu.VMEM((ROWS,), jnp.int32)] * 2,
        [pltpu.VMEM((ROWS,), jnp.int32)] * 2,
        [pltpu.VMEM((CB,), jnp.int32)] * 2,
        [pltpu.VMEM((ROWS, D), jnp.bfloat16)] * 2,
        [pltpu.VMEM((ROWS, D), jnp.bfloat16)] * 2,
        [pltpu.VMEM((CB, D), jnp.bfloat16)] * 2,
        [pltpu.VMEM((CB, 32), jnp.bfloat16)] * 2,
        [pltpu.VMEM((CB, 32), jnp.bfloat16)] * 2,
        [pltpu.SemaphoreType.DMA] * 2,
    ],
)(_sc_body)


def _tc_body(p_ref, n_ref, o_ref):
  ones = jnp.ones((32, 8), jnp.bfloat16)
  pos = lax.dot_general(p_ref[...], ones, (((1,), (0,)), ((), ())),
                        preferred_element_type=jnp.float32)[:, :1]
  neg = lax.dot_general(n_ref[...], ones, (((1,), (0,)), ((), ())),
                        preferred_element_type=jnp.float32)[:, :1]
  pos = jnp.clip(pos, -10.0, 10.0)
  neg = jnp.clip(neg, -10.0, 10.0)
  loss = jnp.log1p(jnp.exp(-pos)) + jnp.log1p(jnp.exp(neg))
  o_ref[...] = (jnp.sum(loss) / B).reshape(1, 1)


_tc_call = pl.pallas_call(
    _tc_body,
    out_shape=jax.ShapeDtypeStruct((1, 1), jnp.float32),
)


@jax.jit
def kernel(target, contexts, negatives, w_emb, C_emb):
  target = jnp.asarray(target, jnp.int32)
  ctx_flat = jnp.asarray(contexts, jnp.int32).reshape(B * L)
  neg_flat = jnp.asarray(negatives, jnp.int32).reshape(B * L)
  w16 = w_emb.astype(jnp.bfloat16)
  c16 = C_emb.astype(jnp.bfloat16)
  pos_part, neg_part = _sc_call(target, ctx_flat, neg_flat, w16, c16)
  return _tc_call(pos_part, neg_part)[0, 0]
